# baseline (speedup 1.0000x reference)
import functools

import jax
import jax.numpy as jnp
from jax import lax
from jax.experimental import pallas as pl
from jax.experimental.pallas import tpu as pltpu
from jax.experimental.pallas import tpu_sc as plsc

N = 10000
EMB = 128
E = 320000
NL = 4
NW = 32
EPW = E // NW
SW = 2000
NSW = EPW // SW
CH = 80
NCH = SW // CH
BN = 1000


def _tc_project_body(e_ref, m_ref, we_ref, wm_ref, bc_ref, ws_ref, wt_ref,
                     bh_ref, upd_ref, u_ref, v_ref):
    u = jnp.tanh(
        jnp.dot(e_ref[...], we_ref[...], preferred_element_type=jnp.float32)
        + jnp.dot(m_ref[...], wm_ref[...], preferred_element_type=jnp.float32)
        + bc_ref[...]
    )
    upd_ref[...] = u
    sb = jnp.dot(u, ws_ref[...], preferred_element_type=jnp.float32) + bh_ref[...]
    tb = jnp.dot(u, wt_ref[...], preferred_element_type=jnp.float32) + bh_ref[...]
    sbits = jax.lax.bitcast_convert_type(sb.astype(jnp.bfloat16), jnp.uint16)
    tbits = jax.lax.bitcast_convert_type(tb.astype(jnp.bfloat16), jnp.uint16)
    for l in range(NL):
        slo = sbits[:, 256 * l:256 * l + 128].astype(jnp.int32)
        shi = sbits[:, 256 * l + 128:256 * (l + 1)].astype(jnp.int32)
        tlo = tbits[:, 256 * l:256 * l + 128].astype(jnp.int32)
        thi = tbits[:, 256 * l + 128:256 * (l + 1)].astype(jnp.int32)
        u_ref[:, 128 * l:128 * (l + 1)] = jax.lax.bitcast_convert_type(
            slo | (shi << 16), jnp.float32)
        v_ref[:, 128 * l:128 * (l + 1)] = jax.lax.bitcast_convert_type(
            tlo | (thi << 16), jnp.float32)


def _tc_project(e, m, we, wm, bc2, ws, wt, bh):
    return pl.pallas_call(
        _tc_project_body,
        grid=(N // BN,),
        in_specs=[
            pl.BlockSpec((BN, EMB), lambda i: (i, 0)),
            pl.BlockSpec((BN, 8), lambda i: (i, 0)),
            pl.BlockSpec((EMB, EMB), lambda i: (0, 0)),
            pl.BlockSpec((8, EMB), lambda i: (0, 0)),
            pl.BlockSpec((1, EMB), lambda i: (0, 0)),
            pl.BlockSpec((EMB, NL * 256), lambda i: (0, 0)),
            pl.BlockSpec((EMB, NL * 256), lambda i: (0, 0)),
            pl.BlockSpec((1, NL * 256), lambda i: (0, 0)),
        ],
        out_specs=[pl.BlockSpec((BN, EMB), lambda i: (i, 0)),
                   pl.BlockSpec((BN, NL * EMB), lambda i: (i, 0)),
                   pl.BlockSpec((BN, NL * EMB), lambda i: (i, 0))],
        out_shape=[jax.ShapeDtypeStruct((N, EMB), jnp.float32),
                   jax.ShapeDtypeStruct((N, NL * EMB), jnp.float32),
                   jax.ShapeDtypeStruct((N, NL * EMB), jnp.float32)],
    )(e, m, we, wm, bc2, ws, wt, bh)


def _tc_combine_body(upd_ref, acc_ref, out_ref):
    a = upd_ref[...] + acc_ref[0] + acc_ref[1]
    nrm = jnp.sqrt(jnp.sum(a * a, axis=1, keepdims=True))
    out_ref[...] = a / jnp.maximum(nrm, 1e-12)


def _tc_combine(upd, acc2):
    return pl.pallas_call(
        _tc_combine_body,
        grid=(N // BN,),
        in_specs=[
            pl.BlockSpec((BN, EMB), lambda i: (i, 0)),
            pl.BlockSpec((2, BN, EMB), lambda i: (0, i, 0)),
        ],
        out_specs=pl.BlockSpec((BN, EMB), lambda i: (i, 0)),
        out_shape=jax.ShapeDtypeStruct((N, EMB), jnp.float32),
    )(upd, acc2)


def _tanh16(x2):
    return 1.0 - 2.0 / (jnp.exp(x2) + 1.0)


def _sc_edge_body(u_hbm, v_hbm, sidx_hbm, oidx_hbm, lay_hbm, zacc_hbm,
                  out_hbm, gsb, gob, ub0, vb0, ub1, vb1, sca0, scb0,
                  sca1, scb1, acc_sh, sem_g0, sem_g1, sem_s0, sem_s1):
    cid = lax.axis_index("c")
    sid = lax.axis_index("s")
    wid = sid * 2 + cid

    rows0 = sid * 624
    pltpu.sync_copy(zacc_hbm.at[pl.ds(rows0, 640)],
                    acc_sh.at[pl.ds(rows0, 640)])
    plsc.subcore_barrier()

    sets = ((ub0, vb0, sca0, scb0, sem_g0, sem_s0),
            (ub1, vb1, sca1, scb1, sem_g1, sem_s1))

    def fire_gathers(c, p):
        ub, vb, _, _, sg, _ = sets[p]
        pltpu.async_copy(u_hbm.at[gsb.at[pl.ds(c * CH, CH)]], ub, sg)
        pltpu.async_copy(v_hbm.at[gob.at[pl.ds(c * CH, CH)]], vb, sg)

    def wait_gathers(p):
        ub, vb, _, _, sg, _ = sets[p]
        pltpu.make_async_copy(u_hbm.at[pl.ds(0, CH)], ub, sg).wait()
        pltpu.make_async_copy(u_hbm.at[pl.ds(0, CH)], vb, sg).wait()

    def wait_scatters(p):
        ub, vb, sa, sb_, _, ss = sets[p]
        pltpu.make_async_copy(ub, acc_sh.at[sa], ss).wait()
        pltpu.make_async_copy(vb, acc_sh.at[sb_], ss).wait()

    def build_scidx(c, p):
        _, _, sa, sb_, _, _ = sets[p]
        for j in range(CH // 16):
            sa[pl.ds(j * 16, 16)] = gsb[pl.ds(c * CH + j * 16, 16)] >> 2
            sb_[pl.ds(j * 16, 16)] = gob[pl.ds(c * CH + j * 16, 16)] >> 2

    def process(c, p, fire_next, cond=None):
        ub, vb, sa, sb_, _, ss = sets[p]
        wait_gathers(p)

        def row2(i, _):
            bci = lambda z: jax.lax.bitcast_convert_type(z, jnp.int32)
            bcf = lambda z: jax.lax.bitcast_convert_type(z, jnp.float32)
            for r in range(2):
                for b in range(EMB // 16):
                    sl = pl.ds(b * 16, 16)
                    uu = bci(ub[2 * i + r, sl])
                    vv = bci(vb[2 * i + r, sl])
                    us = _tanh16(bcf(uu << 16) + bcf(vv << 16))
                    uo = _tanh16(bcf(uu) + bcf(vv))
                    ub[2 * i + r, sl] = us
                    vb[2 * i + r, sl] = uo
            return 0

        lax.fori_loop(0, CH // 4, row2, 0)
        if cond is None:
            wait_scatters(1 - p)
        else:
            @pl.when(cond)
            def _():
                wait_scatters(1 - p)
        if fire_next:
            fire_gathers(c + 1, 1 - p)
            build_scidx(c + 1, 1 - p)
        lax.fori_loop(CH // 4, CH // 2, row2, 0)
        pltpu.async_copy(ub, acc_sh.at[sa], ss, add=True)
        pltpu.async_copy(vb, acc_sh.at[sb_], ss, add=True)

    def sweep(w, _):
        base = wid * EPW + w * SW
        pltpu.sync_copy(sidx_hbm.at[pl.ds(base, SW)], gsb)
        pltpu.sync_copy(lay_hbm.at[pl.ds(base, SW)], gob)

        def build1(k, _):
            sl = pl.ds(k * 16, 16)
            gsb[sl] = gsb[sl] * 4 + gob[sl]
            return 0

        lax.fori_loop(0, SW // 16, build1, 0)
        pltpu.sync_copy(oidx_hbm.at[pl.ds(base, SW)], gob)

        def build2(k, _):
            sl = pl.ds(k * 16, 16)
            gob[sl] = gob[sl] * 4 + (gsb[sl] & 3)
            return 0

        lax.fori_loop(0, SW // 16, build2, 0)

        fire_gathers(0, 0)
        build_scidx(0, 0)

        def dchunk(d, _):
            process(2 * d, 0, True, cond=d > 0)
            process(2 * d + 1, 1, True)
            return 0

        lax.fori_loop(0, (NCH - 1) // 2, dchunk, 0)
        process(NCH - 1, 0, False)
        wait_scatters(0)
        return 0

    lax.fori_loop(0, NSW, sweep, 0)
    plsc.subcore_barrier()

    pltpu.sync_copy(acc_sh.at[pl.ds(rows0, 640)],
                    out_hbm.at[cid, pl.ds(rows0, 640)])


_sc_edge = functools.partial(
    pl.kernel,
    out_type=jax.ShapeDtypeStruct((2, N, EMB), jnp.float32),
    mesh=plsc.VectorSubcoreMesh(core_axis_name="c", subcore_axis_name="s"),
    scratch_types=[
        pltpu.VMEM((SW,), jnp.int32),
        pltpu.VMEM((SW,), jnp.int32),
        pltpu.VMEM((CH, EMB), jnp.float32),
        pltpu.VMEM((CH, EMB), jnp.float32),
        pltpu.VMEM((CH, EMB), jnp.float32),
        pltpu.VMEM((CH, EMB), jnp.float32),
        pltpu.VMEM((CH,), jnp.int32),
        pltpu.VMEM((CH,), jnp.int32),
        pltpu.VMEM((CH,), jnp.int32),
        pltpu.VMEM((CH,), jnp.int32),
        pltpu.VMEM_SHARED((N, EMB), jnp.float32),
        pltpu.SemaphoreType.DMA,
        pltpu.SemaphoreType.DMA,
        pltpu.SemaphoreType.DMA,
        pltpu.SemaphoreType.DMA,
    ],
)(_sc_edge_body)


def kernel(embedding_m, memberships, s_idx, o_idx, layer_id, We, Wm, b_c, Ws,
           bs, Wo, bo):
    w_s = 2.0 * jnp.concatenate(
        [jnp.concatenate([Ws[l, :EMB, :], Wo[l, :EMB, :]], axis=1)
         for l in range(NL)], axis=1)
    w_t = 2.0 * jnp.concatenate(
        [jnp.concatenate([Ws[l, EMB:, :], Wo[l, EMB:, :]], axis=1)
         for l in range(NL)], axis=1)
    bh = jnp.concatenate([bs, bo], axis=1).reshape(1, NL * 256)
    bc2 = b_c.reshape(1, EMB)
    zacc = jnp.zeros((N, EMB), jnp.float32)

    e = embedding_m
    for _t in range(2):
        upd, u_tbl, v_tbl = _tc_project(e, memberships, We, Wm, bc2,
                                        w_s, w_t, bh)
        acc2 = _sc_edge(u_tbl.reshape(N * NL, EMB),
                        v_tbl.reshape(N * NL, EMB),
                        s_idx, o_idx, layer_id, zacc)
        e = _tc_combine(upd, acc2)
    return e


# speedup vs baseline: 2.9080x; 2.9080x over previous
import functools

import jax
import jax.numpy as jnp
from jax import lax
from jax.experimental import pallas as pl
from jax.experimental.pallas import tpu as pltpu
from jax.experimental.pallas import tpu_sc as plsc

N = 10000
EMB = 128
E = 320000
NL = 4
NW = 32
EPW = E // NW
SW = 2000
NSW = EPW // SW
CH = 80
NCH = SW // CH
BN = 1000


def _tc_project_core(e, m_ref, we_ref, wm_ref, bc_ref, ws_ref, wt_ref,
                     bh_ref, upd_ref, u_ref, v_ref):
    u = jnp.tanh(
        jnp.dot(e, we_ref[...], preferred_element_type=jnp.float32)
        + jnp.dot(m_ref[...], wm_ref[...], preferred_element_type=jnp.float32)
        + bc_ref[...]
    )
    upd_ref[...] = u
    sb = jnp.dot(u, ws_ref[...], preferred_element_type=jnp.float32) + bh_ref[...]
    tb = jnp.dot(u, wt_ref[...], preferred_element_type=jnp.float32) + bh_ref[...]
    sbits = jax.lax.bitcast_convert_type(sb.astype(jnp.bfloat16), jnp.uint16)
    tbits = jax.lax.bitcast_convert_type(tb.astype(jnp.bfloat16), jnp.uint16)
    for l in range(NL):
        slo = sbits[:, 256 * l:256 * l + 128].astype(jnp.int32)
        shi = sbits[:, 256 * l + 128:256 * (l + 1)].astype(jnp.int32)
        tlo = tbits[:, 256 * l:256 * l + 128].astype(jnp.int32)
        thi = tbits[:, 256 * l + 128:256 * (l + 1)].astype(jnp.int32)
        u_ref[:, 128 * l:128 * (l + 1)] = jax.lax.bitcast_convert_type(
            slo | (shi << 16), jnp.float32)
        v_ref[:, 128 * l:128 * (l + 1)] = jax.lax.bitcast_convert_type(
            tlo | (thi << 16), jnp.float32)


def _tc_project_body(e_ref, m_ref, we_ref, wm_ref, bc_ref, ws_ref, wt_ref,
                     bh_ref, upd_ref, u_ref, v_ref):
    _tc_project_core(e_ref[...], m_ref, we_ref, wm_ref, bc_ref, ws_ref,
                     wt_ref, bh_ref, upd_ref, u_ref, v_ref)


def _tc_project(e, m, we, wm, bc2, ws, wt, bh):
    return pl.pallas_call(
        _tc_project_body,
        grid=(N // BN,),
        in_specs=[
            pl.BlockSpec((BN, EMB), lambda i: (i, 0)),
            pl.BlockSpec((BN, 8), lambda i: (i, 0)),
            pl.BlockSpec((EMB, EMB), lambda i: (0, 0)),
            pl.BlockSpec((8, EMB), lambda i: (0, 0)),
            pl.BlockSpec((1, EMB), lambda i: (0, 0)),
            pl.BlockSpec((EMB, NL * 256), lambda i: (0, 0)),
            pl.BlockSpec((EMB, NL * 256), lambda i: (0, 0)),
            pl.BlockSpec((1, NL * 256), lambda i: (0, 0)),
        ],
        out_specs=[pl.BlockSpec((BN, EMB), lambda i: (i, 0)),
                   pl.BlockSpec((BN, NL * EMB), lambda i: (i, 0)),
                   pl.BlockSpec((BN, NL * EMB), lambda i: (i, 0))],
        out_shape=[jax.ShapeDtypeStruct((N, EMB), jnp.float32),
                   jax.ShapeDtypeStruct((N, NL * EMB), jnp.float32),
                   jax.ShapeDtypeStruct((N, NL * EMB), jnp.float32)],
    )(e, m, we, wm, bc2, ws, wt, bh)


def _tc_fused_body(updp_ref, acc_ref, m_ref, we_ref, wm_ref, bc_ref,
                   ws_ref, wt_ref, bh_ref, upd_ref, u_ref, v_ref):
    a = updp_ref[...] + acc_ref[0] + acc_ref[1]
    nrm = jnp.sqrt(jnp.sum(a * a, axis=1, keepdims=True))
    e = a / jnp.maximum(nrm, 1e-12)
    _tc_project_core(e, m_ref, we_ref, wm_ref, bc_ref, ws_ref, wt_ref,
                     bh_ref, upd_ref, u_ref, v_ref)


def _tc_fused(updp, acc2, m, we, wm, bc2, ws, wt, bh):
    return pl.pallas_call(
        _tc_fused_body,
        grid=(N // BN,),
        in_specs=[
            pl.BlockSpec((BN, EMB), lambda i: (i, 0)),
            pl.BlockSpec((2, BN, EMB), lambda i: (0, i, 0)),
            pl.BlockSpec((BN, 8), lambda i: (i, 0)),
            pl.BlockSpec((EMB, EMB), lambda i: (0, 0)),
            pl.BlockSpec((8, EMB), lambda i: (0, 0)),
            pl.BlockSpec((1, EMB), lambda i: (0, 0)),
            pl.BlockSpec((EMB, NL * 256), lambda i: (0, 0)),
            pl.BlockSpec((EMB, NL * 256), lambda i: (0, 0)),
            pl.BlockSpec((1, NL * 256), lambda i: (0, 0)),
        ],
        out_specs=[pl.BlockSpec((BN, EMB), lambda i: (i, 0)),
                   pl.BlockSpec((BN, NL * EMB), lambda i: (i, 0)),
                   pl.BlockSpec((BN, NL * EMB), lambda i: (i, 0))],
        out_shape=[jax.ShapeDtypeStruct((N, EMB), jnp.float32),
                   jax.ShapeDtypeStruct((N, NL * EMB), jnp.float32),
                   jax.ShapeDtypeStruct((N, NL * EMB), jnp.float32)],
    )(updp, acc2, m, we, wm, bc2, ws, wt, bh)


def _tc_combine_body(upd_ref, acc_ref, out_ref):
    a = upd_ref[...] + acc_ref[0] + acc_ref[1]
    nrm = jnp.sqrt(jnp.sum(a * a, axis=1, keepdims=True))
    out_ref[...] = a / jnp.maximum(nrm, 1e-12)


def _tc_combine(upd, acc2):
    return pl.pallas_call(
        _tc_combine_body,
        grid=(N // BN,),
        in_specs=[
            pl.BlockSpec((BN, EMB), lambda i: (i, 0)),
            pl.BlockSpec((2, BN, EMB), lambda i: (0, i, 0)),
        ],
        out_specs=pl.BlockSpec((BN, EMB), lambda i: (i, 0)),
        out_shape=jax.ShapeDtypeStruct((N, EMB), jnp.float32),
    )(upd, acc2)


def _tanh16(x2):
    return 1.0 - 2.0 / (jnp.exp(x2) + 1.0)


def _sc_edge_body(u_hbm, v_hbm, sidx_hbm, oidx_hbm, lay_hbm, zacc_hbm,
                  out_hbm, gsb, gob, ub0, vb0, ub1, vb1, sca0, scb0,
                  sca1, scb1, acc_sh, sem_g0, sem_g1, sem_s0, sem_s1):
    cid = lax.axis_index("c")
    sid = lax.axis_index("s")
    wid = sid * 2 + cid

    rows0 = sid * 624
    pltpu.sync_copy(zacc_hbm.at[pl.ds(rows0, 640)],
                    acc_sh.at[pl.ds(rows0, 640)])
    plsc.subcore_barrier()

    sets = ((ub0, vb0, sca0, scb0, sem_g0, sem_s0),
            (ub1, vb1, sca1, scb1, sem_g1, sem_s1))

    def fire_gathers(c, p):
        ub, vb, _, _, sg, _ = sets[p]
        pltpu.async_copy(u_hbm.at[gsb.at[pl.ds(c * CH, CH)]], ub, sg)
        pltpu.async_copy(v_hbm.at[gob.at[pl.ds(c * CH, CH)]], vb, sg)

    def wait_gathers(p):
        ub, vb, _, _, sg, _ = sets[p]
        pltpu.make_async_copy(u_hbm.at[pl.ds(0, CH)], ub, sg).wait()
        pltpu.make_async_copy(u_hbm.at[pl.ds(0, CH)], vb, sg).wait()

    def wait_scatters(p):
        ub, vb, sa, sb_, _, ss = sets[p]
        pltpu.make_async_copy(ub, acc_sh.at[sa], ss).wait()
        pltpu.make_async_copy(vb, acc_sh.at[sb_], ss).wait()

    def process(c, p, fire_next, cond=None):
        ub, vb, sa, sb_, _, ss = sets[p]
        wait_gathers(p)
        for j in range(CH // 16):
            sa[pl.ds(j * 16, 16)] = gsb[pl.ds(c * CH + j * 16, 16)] >> 2
            sb_[pl.ds(j * 16, 16)] = gob[pl.ds(c * CH + j * 16, 16)] >> 2

        def row(i, _):
            bci = lambda z: jax.lax.bitcast_convert_type(z, jnp.int32)
            bcf = lambda z: jax.lax.bitcast_convert_type(z, jnp.float32)
            for b in range(EMB // 16):
                sl = pl.ds(b * 16, 16)
                uu = bci(ub[i, sl])
                vv = bci(vb[i, sl])
                us = _tanh16(bcf(uu << 16) + bcf(vv << 16))
                uo = _tanh16(bcf(uu) + bcf(vv))
                ub[i, sl] = us
                vb[i, sl] = uo
            return 0

        lax.fori_loop(0, CH // 2, row, 0)
        if cond is None:
            wait_scatters(1 - p)
            if fire_next:
                fire_gathers(c + 1, 1 - p)
        else:
            @pl.when(cond)
            def _():
                wait_scatters(1 - p)

            if fire_next:
                fire_gathers(c + 1, 1 - p)
        lax.fori_loop(CH // 2, CH, row, 0)
        pltpu.async_copy(ub, acc_sh.at[sa], ss, add=True)
        pltpu.async_copy(vb, acc_sh.at[sb_], ss, add=True)

    def sweep(w, _):
        base = wid * EPW + w * SW
        pltpu.sync_copy(sidx_hbm.at[pl.ds(base, SW)], gsb)
        pltpu.sync_copy(lay_hbm.at[pl.ds(base, SW)], gob)

        def build1(k, _):
            sl = pl.ds(k * 16, 16)
            gsb[sl] = gsb[sl] * 4 + gob[sl]
            return 0

        lax.fori_loop(0, SW // 16, build1, 0)
        pltpu.sync_copy(oidx_hbm.at[pl.ds(base, SW)], gob)

        def build2(k, _):
            sl = pl.ds(k * 16, 16)
            gob[sl] = gob[sl] * 4 + (gsb[sl] & 3)
            return 0

        lax.fori_loop(0, SW // 16, build2, 0)

        fire_gathers(0, 0)

        def dchunk(d, _):
            process(2 * d, 0, True, cond=d > 0)
            process(2 * d + 1, 1, True)
            return 0

        lax.fori_loop(0, (NCH - 1) // 2, dchunk, 0)
        process(NCH - 1, 0, False)
        wait_scatters(0)
        return 0

    lax.fori_loop(0, NSW, sweep, 0)
    plsc.subcore_barrier()

    pltpu.sync_copy(acc_sh.at[pl.ds(rows0, 640)],
                    out_hbm.at[cid, pl.ds(rows0, 640)])


_sc_edge = functools.partial(
    pl.kernel,
    out_type=jax.ShapeDtypeStruct((2, N, EMB), jnp.float32),
    mesh=plsc.VectorSubcoreMesh(core_axis_name="c", subcore_axis_name="s"),
    scratch_types=[
        pltpu.VMEM((SW,), jnp.int32),
        pltpu.VMEM((SW,), jnp.int32),
        pltpu.VMEM((CH, EMB), jnp.float32),
        pltpu.VMEM((CH, EMB), jnp.float32),
        pltpu.VMEM((CH, EMB), jnp.float32),
        pltpu.VMEM((CH, EMB), jnp.float32),
        pltpu.VMEM((CH,), jnp.int32),
        pltpu.VMEM((CH,), jnp.int32),
        pltpu.VMEM((CH,), jnp.int32),
        pltpu.VMEM((CH,), jnp.int32),
        pltpu.VMEM_SHARED((N, EMB), jnp.float32),
        pltpu.SemaphoreType.DMA,
        pltpu.SemaphoreType.DMA,
        pltpu.SemaphoreType.DMA,
        pltpu.SemaphoreType.DMA,
    ],
)(_sc_edge_body)


def kernel(embedding_m, memberships, s_idx, o_idx, layer_id, We, Wm, b_c, Ws,
           bs, Wo, bo):
    w_s = 2.0 * jnp.concatenate(
        [jnp.concatenate([Ws[l, :EMB, :], Wo[l, :EMB, :]], axis=1)
         for l in range(NL)], axis=1)
    w_t = 2.0 * jnp.concatenate(
        [jnp.concatenate([Ws[l, EMB:, :], Wo[l, EMB:, :]], axis=1)
         for l in range(NL)], axis=1)
    bh = jnp.concatenate([bs, bo], axis=1).reshape(1, NL * 256)
    bc2 = b_c.reshape(1, EMB)
    zacc = jnp.zeros((N, EMB), jnp.float32)

    upd, u_tbl, v_tbl = _tc_project(embedding_m, memberships, We, Wm, bc2,
                                    w_s, w_t, bh)
    acc2 = _sc_edge(u_tbl.reshape(N * NL, EMB), v_tbl.reshape(N * NL, EMB),
                    s_idx, o_idx, layer_id, zacc)
    upd, u_tbl, v_tbl = _tc_fused(upd, acc2, memberships, We, Wm, bc2,
                                  w_s, w_t, bh)
    acc2 = _sc_edge(u_tbl.reshape(N * NL, EMB), v_tbl.reshape(N * NL, EMB),
                    s_idx, o_idx, layer_id, zacc)
    return _tc_combine(upd, acc2)
